# baseline (speedup 1.0000x reference)
import functools

import jax
import jax.numpy as jnp
from jax import lax
from jax.experimental import pallas as pl
from jax.experimental.pallas import tpu as pltpu
from jax.experimental.pallas import tpu_sc as plsc

_B, _L, _V, _D, _NPOS = 4096, 200, 100000, 128, 201
_N = _B * _L
_NC, _NS = 2, 16
_NW = _NC * _NS
_PER_W = _N // _NW
_C = 128
_CHUNKS = _PER_W // _C
_T = _CHUNKS // 2
_LANES = 16
_ADD_UNROLL = 1


def _bcast_lane(vec, i):
    idx = jnp.full((_LANES, 1), i, jnp.int32)
    return lax.gather(
        vec, idx,
        dimension_numbers=lax.GatherDimensionNumbers(
            offset_dims=(), collapsed_slice_dims=(0,), start_index_map=(0,)),
        slice_sizes=(1,),
        mode=lax.GatherScatterMode.PROMISE_IN_BOUNDS)


def _emb_body(seq_ref, pos_ref, wtab_ref, ptab_ref, out_ref,
              widx0, widx1, pidx0, pidx1, buf0, buf1, ptab_v,
              sem_i0, sem_i1, sem_g0, sem_g1, sem_s0, sem_s1):
    widx = (widx0, widx1)
    pidx = (pidx0, pidx1)
    buf = (buf0, buf1)
    sem_i = (sem_i0, sem_i1)
    sem_g = (sem_g0, sem_g1)
    sem_s = (sem_s0, sem_s1)

    wid = lax.axis_index("s") * _NC + lax.axis_index("c")
    base = wid * _PER_W
    pltpu.sync_copy(ptab_ref, ptab_v)

    def issue_idx(k, s):
        cb = base + k * _C
        pltpu.async_copy(seq_ref.at[pl.ds(cb, _C)], widx[s], sem_i[s])
        pltpu.async_copy(pos_ref.at[pl.ds(cb, _C)], pidx[s], sem_i[s])

    def wait_idx(s):
        pltpu.make_async_copy(seq_ref.at[pl.ds(base, _C)], widx[s], sem_i[s]).wait()
        pltpu.make_async_copy(pos_ref.at[pl.ds(base, _C)], pidx[s], sem_i[s]).wait()

    def issue_gather(s):
        pltpu.async_copy(wtab_ref.at[widx[s]], buf[s], sem_g[s])

    def wait_gather(s):
        pltpu.make_async_copy(wtab_ref.at[widx[s]], buf[s], sem_g[s]).wait()

    def issue_scatter(k, s):
        pltpu.async_copy(buf[s], out_ref.at[pl.ds(base + k * _C, _C)], sem_s[s])

    def wait_scatter(s):
        pltpu.make_async_copy(buf[s], out_ref.at[pl.ds(base, _C)], sem_s[s]).wait()

    def add_chunk(s):
        pidx_s, buf_s = pidx[s], buf[s]

        @plsc.parallel_loop(0, _C // _LANES, step=1, unroll=_ADD_UNROLL)
        def group_body(g):
            pvec = pidx_s[pl.ds(g * _LANES, _LANES)]
            for i in range(_LANES):
                r = g * _LANES + i
                psplat = _bcast_lane(pvec, i)
                pvs = []
                for j in range(_D // _LANES):
                    cols = lax.iota(jnp.int32, _LANES) + (j * _LANES)
                    pvs.append(plsc.load_gather(ptab_v, [psplat, cols]))
                for j in range(_D // _LANES):
                    plsc.addupdate(buf_s.at[r, pl.ds(j * _LANES, _LANES)], pvs[j])

    issue_idx(0, 0)
    issue_idx(1, 1)
    wait_idx(0)
    issue_gather(0)

    def pipe_body(t, carry):
        k0 = 2 * t
        k1 = 2 * t + 1
        wait_idx(1)
        jax.lax.cond(t > 0, lambda: wait_scatter(1), lambda: None)
        issue_gather(1)
        wait_gather(0)
        add_chunk(0)
        jax.lax.cond(t < _T - 1, lambda: issue_idx(k0 + 2, 0), lambda: None)
        issue_scatter(k0, 0)
        wait_gather(1)
        add_chunk(1)
        jax.lax.cond(t < _T - 1, lambda: issue_idx(k1 + 2, 1), lambda: None)
        issue_scatter(k1, 1)

        def tail():
            wait_idx(0)
            wait_scatter(0)
            issue_gather(0)

        jax.lax.cond(t < _T - 1, tail, lambda: None)
        return carry

    lax.fori_loop(0, _T, pipe_body, 0)
    wait_scatter(0)
    wait_scatter(1)


@jax.jit
def _emb(seq_flat, pos_flat, word_emb, pos_enc):
    mesh = plsc.VectorSubcoreMesh(core_axis_name="c", subcore_axis_name="s")
    run = pl.kernel(
        _emb_body,
        mesh=mesh,
        compiler_params=pltpu.CompilerParams(needs_layout_passes=False),
        out_type=jax.ShapeDtypeStruct((_N, _D), jnp.float32),
        scratch_types=[
            pltpu.VMEM((_C,), jnp.int32),
            pltpu.VMEM((_C,), jnp.int32),
            pltpu.VMEM((_C,), jnp.int32),
            pltpu.VMEM((_C,), jnp.int32),
            pltpu.VMEM((_C, _D), jnp.float32),
            pltpu.VMEM((_C, _D), jnp.float32),
            pltpu.VMEM((_NPOS, _D), jnp.float32),
            pltpu.SemaphoreType.DMA,
            pltpu.SemaphoreType.DMA,
            pltpu.SemaphoreType.DMA,
            pltpu.SemaphoreType.DMA,
            pltpu.SemaphoreType.DMA,
            pltpu.SemaphoreType.DMA,
        ],
    )
    return run(seq_flat, pos_flat, word_emb, pos_enc)


def kernel(src_seq, src_pos, word_emb, pos_enc):
    out = _emb(src_seq.reshape(_N), src_pos.reshape(_N), word_emb, pos_enc)
    return out.reshape(_B, _L, _D), src_seq


# speedup vs baseline: 1.2709x; 1.2709x over previous
import jax
import jax.numpy as jnp
from jax import lax
from jax.experimental import pallas as pl
from jax.experimental.pallas import tpu as pltpu
from jax.experimental.pallas import tpu_sc as plsc

_B, _L, _V, _D, _NPOS = 4096, 200, 100000, 128, 201
_N = _B * _L
_NC, _NS = 2, 16
_NW = _NC * _NS
_PER_W = _N // _NW
_C = 128
_CHUNKS = _PER_W // _C
_NSLOT = 4
_T = _CHUNKS // _NSLOT
_LANES = 16


def _bcast_lane(vec, i):
    idx = jnp.full((_LANES, 1), i, jnp.int32)
    return lax.gather(
        vec, idx,
        dimension_numbers=lax.GatherDimensionNumbers(
            offset_dims=(), collapsed_slice_dims=(0,), start_index_map=(0,)),
        slice_sizes=(1,),
        mode=lax.GatherScatterMode.PROMISE_IN_BOUNDS)


def _emb_body(seq_ref, pos_ref, wtab_ref, ptab_ref, out_ref, *scratch):
    widx = scratch[0:4]
    pidx = scratch[4:8]
    buf = scratch[8:12]
    ptab_v = scratch[12]
    sem_i = scratch[13:17]
    sem_g = scratch[17:21]
    sem_s = scratch[21:25]

    wid = lax.axis_index("s") * _NC + lax.axis_index("c")
    base = wid * _PER_W
    pltpu.sync_copy(ptab_ref, ptab_v)

    def issue_idx(k, s):
        cb = base + k * _C
        pltpu.async_copy(seq_ref.at[pl.ds(cb, _C)], widx[s], sem_i[s])
        pltpu.async_copy(pos_ref.at[pl.ds(cb, _C)], pidx[s], sem_i[s])

    def wait_idx(s):
        pltpu.make_async_copy(seq_ref.at[pl.ds(base, _C)], widx[s], sem_i[s]).wait()
        pltpu.make_async_copy(pos_ref.at[pl.ds(base, _C)], pidx[s], sem_i[s]).wait()

    def issue_gather(s):
        pltpu.async_copy(wtab_ref.at[widx[s]], buf[s], sem_g[s])

    def wait_gather(s):
        pltpu.make_async_copy(wtab_ref.at[widx[s]], buf[s], sem_g[s]).wait()

    def issue_scatter(k, s):
        pltpu.async_copy(buf[s], out_ref.at[pl.ds(base + k * _C, _C)], sem_s[s])

    def wait_scatter(s):
        pltpu.make_async_copy(buf[s], out_ref.at[pl.ds(base, _C)], sem_s[s]).wait()

    def add_chunk(s):
        pidx_s, buf_s = pidx[s], buf[s]

        @plsc.parallel_loop(0, _C // _LANES, step=1)
        def group_body(g):
            pvec = pidx_s[pl.ds(g * _LANES, _LANES)]
            for i in range(_LANES):
                r = g * _LANES + i
                psplat = _bcast_lane(pvec, i)
                pvs = []
                for j in range(_D // _LANES):
                    cols = lax.iota(jnp.int32, _LANES) + (j * _LANES)
                    pvs.append(plsc.load_gather(ptab_v, [psplat, cols]))
                for j in range(_D // _LANES):
                    plsc.addupdate(buf_s.at[r, pl.ds(j * _LANES, _LANES)], pvs[j])

    for s in range(_NSLOT):
        issue_idx(s, s)
    wait_idx(0)
    issue_gather(0)
    wait_idx(1)
    issue_gather(1)

    def step(k, u):
        v = (u + 2) % _NSLOT
        wait_gather(u)
        def launch_next_gather():
            wait_idx(v)
            jax.lax.cond(k >= 2, lambda: wait_scatter(v), lambda: None)
            issue_gather(v)
        jax.lax.cond(k + 2 < _CHUNKS, launch_next_gather, lambda: None)
        add_chunk(u)
        jax.lax.cond(k + 4 < _CHUNKS, lambda: issue_idx(k + 4, u), lambda: None)
        issue_scatter(k, u)

    def pipe_body(t, carry):
        for u in range(_NSLOT):
            step(_NSLOT * t + u, u)
        return carry

    lax.fori_loop(0, _T, pipe_body, 0)
    wait_scatter((_CHUNKS - 2) % _NSLOT)
    wait_scatter((_CHUNKS - 1) % _NSLOT)


@jax.jit
def _emb(seq_flat, pos_flat, word_emb, pos_enc):
    mesh = plsc.VectorSubcoreMesh(core_axis_name="c", subcore_axis_name="s")
    run = pl.kernel(
        _emb_body,
        mesh=mesh,
        compiler_params=pltpu.CompilerParams(needs_layout_passes=False),
        out_type=jax.ShapeDtypeStruct((_N, _D), jnp.float32),
        scratch_types=(
            [pltpu.VMEM((_C,), jnp.int32) for _ in range(2 * _NSLOT)]
            + [pltpu.VMEM((_C, _D), jnp.float32) for _ in range(_NSLOT)]
            + [pltpu.VMEM((_NPOS, _D), jnp.float32)]
            + [pltpu.SemaphoreType.DMA for _ in range(3 * _NSLOT)]
        ),
    )
    return run(seq_flat, pos_flat, word_emb, pos_enc)


def kernel(src_seq, src_pos, word_emb, pos_enc):
    out = _emb(src_seq.reshape(_N), src_pos.reshape(_N), word_emb, pos_enc)
    return out.reshape(_B, _L, _D), src_seq


# speedup vs baseline: 1.2734x; 1.0020x over previous
import jax
import jax.numpy as jnp
from jax import lax
from jax.experimental import pallas as pl
from jax.experimental.pallas import tpu as pltpu
from jax.experimental.pallas import tpu_sc as plsc

_B, _L, _V, _D, _NPOS = 4096, 200, 100000, 128, 201
_N = _B * _L
_NC, _NS = 2, 16
_NW = _NC * _NS
_PER_W = _N // _NW
_C = 128
_CHUNKS = _PER_W // _C
_NSLOT = 4
_T = _CHUNKS // _NSLOT
_LANES = 16


def _bcast_lane(vec, i):
    idx = jnp.full((_LANES, 1), i, jnp.int32)
    return lax.gather(
        vec, idx,
        dimension_numbers=lax.GatherDimensionNumbers(
            offset_dims=(), collapsed_slice_dims=(0,), start_index_map=(0,)),
        slice_sizes=(1,),
        mode=lax.GatherScatterMode.PROMISE_IN_BOUNDS)


def _emb_body(seq_ref, pos_ref, wtab_ref, ptab_ref, out_ref, *scratch):
    widx = scratch[0:4]
    pidx = scratch[4:8]
    buf = scratch[8:12]
    ptab_v = scratch[12]
    sem_i = scratch[13:17]
    sem_g = scratch[17:21]
    sem_s = scratch[21:25]
    sem_ptab = scratch[25]

    wid = lax.axis_index("s") * _NC + lax.axis_index("c")
    base = wid * _PER_W
    ptab_copy = pltpu.make_async_copy(ptab_ref, ptab_v, sem_ptab)
    pltpu.async_copy(ptab_ref, ptab_v, sem_ptab)

    def issue_idx(k, s):
        cb = base + k * _C
        pltpu.async_copy(seq_ref.at[pl.ds(cb, _C)], widx[s], sem_i[s])
        pltpu.async_copy(pos_ref.at[pl.ds(cb, _C)], pidx[s], sem_i[s])

    def wait_idx(s):
        pltpu.make_async_copy(seq_ref.at[pl.ds(base, _C)], widx[s], sem_i[s]).wait()
        pltpu.make_async_copy(pos_ref.at[pl.ds(base, _C)], pidx[s], sem_i[s]).wait()

    def issue_gather(s):
        pltpu.async_copy(wtab_ref.at[widx[s]], buf[s], sem_g[s])

    def wait_gather(s):
        pltpu.make_async_copy(wtab_ref.at[widx[s]], buf[s], sem_g[s]).wait()

    def issue_scatter(k, s):
        pltpu.async_copy(buf[s], out_ref.at[pl.ds(base + k * _C, _C)], sem_s[s])

    def wait_scatter(s):
        pltpu.make_async_copy(buf[s], out_ref.at[pl.ds(base, _C)], sem_s[s]).wait()

    def add_chunk(s):
        pidx_s, buf_s = pidx[s], buf[s]

        @plsc.parallel_loop(0, _C // _LANES, step=1)
        def group_body(g):
            pvec = pidx_s[pl.ds(g * _LANES, _LANES)]
            for i in range(_LANES):
                r = g * _LANES + i
                psplat = _bcast_lane(pvec, i)
                pvs = []
                for j in range(_D // _LANES):
                    cols = lax.iota(jnp.int32, _LANES) + (j * _LANES)
                    pvs.append(plsc.load_gather(ptab_v, [psplat, cols]))
                for j in range(_D // _LANES):
                    plsc.addupdate(buf_s.at[r, pl.ds(j * _LANES, _LANES)], pvs[j])

    for s in range(_NSLOT):
        issue_idx(s, s)
    wait_idx(0)
    issue_gather(0)
    wait_idx(1)
    issue_gather(1)
    ptab_copy.wait()

    def step(k, u):
        v = (u + 2) % _NSLOT
        wait_gather(u)
        def launch_next_gather():
            wait_idx(v)
            jax.lax.cond(k >= 2, lambda: wait_scatter(v), lambda: None)
            issue_gather(v)
        jax.lax.cond(k + 2 < _CHUNKS, launch_next_gather, lambda: None)
        add_chunk(u)
        jax.lax.cond(k + 4 < _CHUNKS, lambda: issue_idx(k + 4, u), lambda: None)
        issue_scatter(k, u)

    def pipe_body(t, carry):
        for u in range(_NSLOT):
            step(_NSLOT * t + u, u)
        return carry

    lax.fori_loop(0, _T, pipe_body, 0)
    wait_scatter((_CHUNKS - 2) % _NSLOT)
    wait_scatter((_CHUNKS - 1) % _NSLOT)


@jax.jit
def _emb(seq_flat, pos_flat, word_emb, pos_enc):
    mesh = plsc.VectorSubcoreMesh(core_axis_name="c", subcore_axis_name="s")
    run = pl.kernel(
        _emb_body,
        mesh=mesh,
        compiler_params=pltpu.CompilerParams(needs_layout_passes=False),
        out_type=jax.ShapeDtypeStruct((_N, _D), jnp.float32),
        scratch_types=(
            [pltpu.VMEM((_C,), jnp.int32) for _ in range(2 * _NSLOT)]
            + [pltpu.VMEM((_C, _D), jnp.float32) for _ in range(_NSLOT)]
            + [pltpu.VMEM((_NPOS, _D), jnp.float32)]
            + [pltpu.SemaphoreType.DMA for _ in range(3 * _NSLOT + 1)]
        ),
    )
    return run(seq_flat, pos_flat, word_emb, pos_enc)


def kernel(src_seq, src_pos, word_emb, pos_enc):
    out = _emb(src_seq.reshape(_N), src_pos.reshape(_N), word_emb, pos_enc)
    return out.reshape(_B, _L, _D), src_seq


# speedup vs baseline: 1.2768x; 1.0027x over previous
import jax
import jax.numpy as jnp
from jax import lax
from jax.experimental import pallas as pl
from jax.experimental.pallas import tpu as pltpu
from jax.experimental.pallas import tpu_sc as plsc

_B, _L, _V, _D, _NPOS = 4096, 200, 100000, 128, 201
_N = _B * _L
_NC, _NS = 2, 16
_NW = _NC * _NS
_PER_W = _N // _NW
_C = 128
_CHUNKS = _PER_W // _C
_NSLOT = 4
_T = _CHUNKS // _NSLOT
_LANES = 16


def _bcast_lane(vec, i):
    idx = jnp.full((_LANES, 1), i, jnp.int32)
    return lax.gather(
        vec, idx,
        dimension_numbers=lax.GatherDimensionNumbers(
            offset_dims=(), collapsed_slice_dims=(0,), start_index_map=(0,)),
        slice_sizes=(1,),
        mode=lax.GatherScatterMode.PROMISE_IN_BOUNDS)


def _emb_body(seq_ref, pos_ref, wtab_ref, ptab_ref, out_ref, *scratch):
    widx = scratch[0:4]
    pidx = scratch[4:8]
    buf = scratch[8:12]
    ptab_v = scratch[12]
    sem_i = scratch[13:17]
    sem_g = scratch[17:21]
    sem_s = scratch[21:25]
    sem_ptab = scratch[25]

    wid = lax.axis_index("s") * _NC + lax.axis_index("c")
    base = wid * _PER_W
    ptab_copy = pltpu.make_async_copy(ptab_ref, ptab_v, sem_ptab)
    pltpu.async_copy(ptab_ref, ptab_v, sem_ptab)

    def issue_idx(k, s):
        cb = base + k * _C
        pltpu.async_copy(seq_ref.at[pl.ds(cb, _C)], widx[s], sem_i[s])
        pltpu.async_copy(pos_ref.at[pl.ds(cb, _C)], pidx[s], sem_i[s])

    def wait_idx(s):
        pltpu.make_async_copy(seq_ref.at[pl.ds(base, _C)], widx[s], sem_i[s]).wait()
        pltpu.make_async_copy(pos_ref.at[pl.ds(base, _C)], pidx[s], sem_i[s]).wait()

    def issue_gather(s):
        pltpu.async_copy(wtab_ref.at[widx[s]], buf[s], sem_g[s])

    def wait_gather(s):
        pltpu.make_async_copy(wtab_ref.at[widx[s]], buf[s], sem_g[s]).wait()

    def issue_scatter(k, s):
        pltpu.async_copy(buf[s], out_ref.at[pl.ds(base + k * _C, _C)], sem_s[s])

    def wait_scatter(s):
        pltpu.make_async_copy(buf[s], out_ref.at[pl.ds(base, _C)], sem_s[s]).wait()

    def add_chunk(s):
        pidx_s, buf_s = pidx[s], buf[s]

        @plsc.parallel_loop(0, _C // _LANES, step=1)
        def group_body(g):
            pvec = pidx_s[pl.ds(g * _LANES, _LANES)]
            for i in range(_LANES):
                r = g * _LANES + i
                psplat = _bcast_lane(pvec, i)
                pvs = []
                for j in range(_D // _LANES):
                    cols = lax.iota(jnp.int32, _LANES) + (j * _LANES)
                    pvs.append(plsc.load_gather(ptab_v, [psplat, cols]))
                for j in range(_D // _LANES):
                    plsc.addupdate(buf_s.at[r, pl.ds(j * _LANES, _LANES)], pvs[j])

    for s in range(_NSLOT):
        issue_idx(s, s)
    wait_idx(0)
    issue_gather(0)
    wait_idx(1)
    issue_gather(1)
    ptab_copy.wait()

    def step(k, u):
        v = (u + 2) % _NSLOT
        wait_gather(u)
        def launch_next_gather():
            wait_idx(v)
            jax.lax.cond(k >= 2, lambda: wait_scatter(v), lambda: None)
            issue_gather(v)
        jax.lax.cond(k + 2 < _CHUNKS, launch_next_gather, lambda: None)
        add_chunk(u)
        jax.lax.cond(k + 4 < _CHUNKS, lambda: issue_idx(k + 4, u), lambda: None)
        issue_scatter(k, u)

    def pipe_body(t, carry):
        for u in range(_NSLOT):
            step(_NSLOT * t + u, u)
        return carry

    lax.fori_loop(0, _T, pipe_body, 0)
    wait_scatter((_CHUNKS - 2) % _NSLOT)
    wait_scatter((_CHUNKS - 1) % _NSLOT)


@jax.jit
def _emb(seq_flat, pos_flat, word_emb, pos_enc):
    mesh = plsc.VectorSubcoreMesh(core_axis_name="c", subcore_axis_name="s",
                                  num_cores=_NC, num_subcores=_NS)
    run = pl.kernel(
        _emb_body,
        mesh=mesh,
        compiler_params=pltpu.CompilerParams(needs_layout_passes=False),
        out_type=jax.ShapeDtypeStruct((_N, _D), jnp.float32),
        scratch_types=(
            [pltpu.VMEM((_C,), jnp.int32) for _ in range(2 * _NSLOT)]
            + [pltpu.VMEM((_C, _D), jnp.float32) for _ in range(_NSLOT)]
            + [pltpu.VMEM((_NPOS, _D), jnp.float32)]
            + [pltpu.SemaphoreType.DMA for _ in range(3 * _NSLOT + 1)]
        ),
    )
    return run(seq_flat, pos_flat, word_emb, pos_enc)


def kernel(src_seq, src_pos, word_emb, pos_enc):
    out = _emb(src_seq.reshape(_N), src_pos.reshape(_N), word_emb, pos_enc)
    return out.reshape(_B, _L, _D), src_seq
